# Initial kernel scaffold; baseline (speedup 1.0000x reference)
import jax
import jax.numpy as jnp
from jax.experimental import pallas as pl


def kernel(x, edge_index, batch, att, W0, b0, W1, b1, W2, b2):
    raise NotImplementedError("write your pallas kernel here")


# speedup vs baseline: 30.5550x; 30.5550x over previous
import functools

import jax
import jax.numpy as jnp
from jax import lax
from jax.experimental import pallas as pl
from jax.experimental.pallas import tpu as pltpu
from jax.experimental.pallas import tpu_sc as plsc

_N = 50000
_E = 800000
_DIN = 128
_HID = 64
_HALF = _HID // 2
_NCLS = 40
_K = 3

_NC = 2
_NS = 16
_G = 80
_NG = 10000
_EPAD = _NG * _G
_NGT = _NG // _NS
_NBUF = 8
_PF = 2
_LAG = 5
_NRND = -(-(_NGT + _PF + _LAG) // _NBUF)

_RCH = 80
_NCH = _N // _RCH
_CPT = -(-_NCH // _NS)
_DW = 16

_RB = 2000
_TCGRID = _N // _RB


def _mesh():
    return plsc.VectorSubcoreMesh(core_axis_name="c", subcore_axis_name="s")


_DNB = 8
_DPF = 4


def _sc_deg(row0, row1):
    @functools.partial(
        pl.kernel,
        mesh=_mesh(),
        compiler_params=pltpu.CompilerParams(use_tc_tiling_on_sc=False),
        out_type=jax.ShapeDtypeStruct((_NC * _N, _DW), jnp.float32),
        scratch_types=[
            pltpu.VMEM((_DNB, 1, _G), jnp.int32),
            pltpu.VMEM((_G, _DW), jnp.float32),
            pltpu.VMEM((_RCH, _DW), jnp.float32),
            pltpu.VMEM_SHARED((_N + 8, _DW), jnp.float32),
        ]
        + [pltpu.SemaphoreType.DMA] * (2 * _DNB),
    )
    def k(r0_h, r1_h, out_h, rowx, ones_v, tbuf, acc, *sems):
        sem_i = sems[:_DNB]
        sem_s = sems[_DNB:]
        cid = lax.axis_index("c")
        sid = lax.axis_index("s")

        def wait_idx(b):
            pltpu.make_async_copy(r0_h.at[0], rowx.at[b], sem_i[b]).wait()

        def wait_scatter(b):
            pltpu.make_async_copy(
                ones_v, acc.at[rowx.at[b, 0]], sem_s[b]).wait()

        @pl.loop(0, _G)
        def _(i):
            ones_v[i, :] = jnp.ones((_DW,), jnp.float32)

        @pl.loop(0, _RCH)
        def _(i):
            tbuf[i, :] = jnp.zeros((_DW,), jnp.float32)

        @pl.loop(0, _CPT)
        def _(i):
            ch = sid + _NS * i

            @pl.when(ch < _NCH)
            def _():
                pltpu.sync_copy(tbuf, acc.at[pl.ds(ch * _RCH, _RCH)])

        plsc.subcore_barrier()

        def count(e_h):
            base = sid * _NGT
            nrnd = -(-(_NGT + _DPF) // _DNB)

            @pl.loop(0, nrnd)
            def _(r):
                for step in range(_DNB):
                    v = r * _DNB + step
                    bi = step
                    bs = (step - _DPF) % _DNB
                    gs = v - _DPF

                    @pl.when(v < _NGT)
                    def _():
                        @pl.when(v >= _DNB)
                        def _():
                            wait_scatter(bi)

                        pltpu.async_copy(e_h.at[base + v],
                                         rowx.at[bi], sem_i[bi])

                    @pl.when((gs >= 0) & (gs < _NGT))
                    def _():
                        wait_idx(bs)
                        pltpu.async_copy(ones_v, acc.at[rowx.at[bs, 0]],
                                         sem_s[bs], add=True)

            for b in range(_DNB):
                wait_scatter(b)

        @pl.when(cid == 0)
        def _():
            count(r0_h)

        @pl.when(cid == 1)
        def _():
            count(r1_h)

        plsc.subcore_barrier()

        @pl.loop(0, _CPT)
        def _(i):
            ch = sid + _NS * i

            @pl.when(ch < _NCH)
            def _():
                pltpu.sync_copy(acc.at[pl.ds(ch * _RCH, _RCH)], tbuf)
                pltpu.sync_copy(
                    tbuf, out_h.at[pl.ds(cid * _N + ch * _RCH, _RCH)])

    return k(row0, row1)


def _sc_prop3(row0, col0, row1, col1, s_lo, s_hi, rdeg):
    @functools.partial(
        pl.kernel,
        mesh=_mesh(),
        compiler_params=pltpu.CompilerParams(use_tc_tiling_on_sc=False),
        out_type=[jax.ShapeDtypeStruct((_N, _HALF), jnp.float32)] * 6,
        scratch_types=[
            pltpu.VMEM((_NBUF, 1, _G), jnp.int32),
            pltpu.VMEM((_NBUF, 1, _G), jnp.int32),
            pltpu.VMEM((_NBUF, _G, _HALF), jnp.float32),
            pltpu.VMEM((_RCH, _HALF), jnp.float32),
            pltpu.VMEM((_RCH, _DW), jnp.float32),
            pltpu.VMEM_SHARED((_N + 8, _HALF), jnp.float32),
        ]
        + [pltpu.SemaphoreType.DMA] * (3 * _NBUF),
    )
    def k(r0_h, c0_h, r1_h, c1_h, slo_h, shi_h, rdeg_h,
          o1l, o1h, o2l, o2h, o3l, o3h,
          colx, rowx, data, tbuf, dbuf, acc, *sems):
        sem_i = sems[:_NBUF]
        sem_g = sems[_NBUF:2 * _NBUF]
        sem_s = sems[2 * _NBUF:]
        cid = lax.axis_index("c")
        sid = lax.axis_index("s")

        def wait_idx(b):
            pltpu.make_async_copy(c0_h.at[0], colx.at[b], sem_i[b]).wait()
            pltpu.make_async_copy(c0_h.at[0], rowx.at[b], sem_i[b]).wait()

        def wait_gather(s_h, b):
            pltpu.make_async_copy(
                s_h.at[colx.at[b, 0]], data.at[b], sem_g[b]).wait()

        def wait_scatter(b):
            pltpu.make_async_copy(
                data.at[b], acc.at[rowx.at[b, 0]], sem_s[b]).wait()

        def seed(s_h):
            @pl.loop(0, _CPT)
            def _(i):
                ch = sid + _NS * i

                @pl.when(ch < _NCH)
                def _():
                    pltpu.sync_copy(s_h.at[pl.ds(ch * _RCH, _RCH)], tbuf)
                    pltpu.sync_copy(tbuf, acc.at[pl.ds(ch * _RCH, _RCH)])

        def edges(s_h):
            base = sid * _NGT
            for row_h, col_h in ((r0_h, c0_h), (r1_h, c1_h)):
                @pl.loop(0, _NRND)
                def _(r):
                    for step in range(_NBUF):
                        v = r * _NBUF + step
                        bi = step
                        ba = (step - _PF) % _NBUF
                        bs = (step - _PF - _LAG) % _NBUF
                        ga = v - _PF
                        gs = v - _PF - _LAG

                        @pl.when(v < _NGT)
                        def _():
                            @pl.when(v >= _NBUF)
                            def _():
                                wait_scatter(bi)

                            pltpu.async_copy(col_h.at[base + v],
                                             colx.at[bi], sem_i[bi])
                            pltpu.async_copy(row_h.at[base + v],
                                             rowx.at[bi], sem_i[bi])

                        @pl.when((ga >= 0) & (ga < _NGT))
                        def _():
                            wait_idx(ba)
                            pltpu.async_copy(s_h.at[colx.at[ba, 0]],
                                             data.at[ba], sem_g[ba])

                        @pl.when((gs >= 0) & (gs < _NGT))
                        def _():
                            wait_gather(s_h, bs)
                            pltpu.async_copy(data.at[bs],
                                             acc.at[rowx.at[bs, 0]],
                                             sem_s[bs], add=True)

                for b in range(_NBUF):
                    wait_scatter(b)

        def drain_seed(o_h, last):
            @pl.loop(0, _CPT)
            def _(i):
                ch = sid + _NS * i

                @pl.when(ch < _NCH)
                def _():
                    r0 = ch * _RCH
                    pltpu.sync_copy(acc.at[pl.ds(r0, _RCH)], tbuf)
                    pltpu.sync_copy(rdeg_h.at[pl.ds(r0, _RCH)], dbuf)

                    @pl.loop(0, _RCH)
                    def _(r):
                        w = dbuf[r, :]
                        tbuf[r, pl.ds(0, 16)] = tbuf[r, pl.ds(0, 16)] * w
                        tbuf[r, pl.ds(16, 16)] = tbuf[r, pl.ds(16, 16)] * w

                    pltpu.sync_copy(tbuf, o_h.at[pl.ds(r0, _RCH)])
                    if not last:
                        pltpu.sync_copy(tbuf, acc.at[pl.ds(r0, _RCH)])

        @pl.when(cid == 0)
        def _():
            seed(slo_h)

        @pl.when(cid == 1)
        def _():
            seed(shi_h)

        plsc.subcore_barrier()

        srcs_lo = [slo_h, o1l, o2l]
        srcs_hi = [shi_h, o1h, o2h]
        outs_lo = [o1l, o2l, o3l]
        outs_hi = [o1h, o2h, o3h]
        for kk in range(3):
            @pl.when(cid == 0)
            def _():
                edges(srcs_lo[kk])

            @pl.when(cid == 1)
            def _():
                edges(srcs_hi[kk])

            plsc.subcore_barrier()

            @pl.when(cid == 0)
            def _():
                drain_seed(outs_lo[kk], kk == 2)

            @pl.when(cid == 1)
            def _():
                drain_seed(outs_hi[kk], kk == 2)

            if kk < 2:
                plsc.subcore_barrier()

    return k(row0, col0, row1, col1, s_lo, s_hi, rdeg)


def _row_spec(width):
    return pl.BlockSpec((_RB, width), lambda i: (i, 0))


def _full_spec(shape):
    return pl.BlockSpec(shape, lambda i: tuple(0 for _ in shape))


def _tc_first(x, W0, b0, p0, p1):
    def body(x_r, w_r, b_r, p0_r, p1_r, h_r, slo_r, shi_r, rdeg_r):
        deg = p0_r[:, :1] + p1_r[:, :1] + 1.0
        dinv = lax.rsqrt(deg)
        h = jnp.dot(x_r[...], w_r[...], preferred_element_type=jnp.float32)
        h = h + b_r[...]
        h_r[...] = h
        s = h * dinv
        slo_r[...] = s[:, :_HALF]
        shi_r[...] = s[:, _HALF:]
        rdeg_r[...] = jnp.broadcast_to(1.0 / deg, (_RB, _DW))

    return pl.pallas_call(
        body,
        grid=(_TCGRID,),
        in_specs=[
            _row_spec(_DIN),
            _full_spec((_DIN, _HID)),
            _full_spec((1, _HID)),
            _row_spec(_DW),
            _row_spec(_DW),
        ],
        out_specs=[
            _row_spec(_HID),
            _row_spec(_HALF),
            _row_spec(_HALF),
            _row_spec(_DW),
        ],
        out_shape=[
            jax.ShapeDtypeStruct((_N, _HID), jnp.float32),
            jax.ShapeDtypeStruct((_N, _HALF), jnp.float32),
            jax.ShapeDtypeStruct((_N, _HALF), jnp.float32),
            jax.ShapeDtypeStruct((_N, _DW), jnp.float32),
        ],
    )(x, W0, b0.reshape(1, _HID), p0, p1)


def _combine(h_prev, s_list, att_row_r, p0_r, p1_r):
    deg = p0_r[:, :1] + p1_r[:, :1] + 1.0
    dsqrt = jnp.sqrt(deg)
    agg = att_row_r[0, 0] * h_prev
    for kk, (lo_r, hi_r) in enumerate(s_list):
        cur = jnp.concatenate([lo_r[...], hi_r[...]], axis=1) * dsqrt
        agg = agg + att_row_r[0, kk + 1] * cur
    return jax.nn.relu(agg), deg


def _tc_mid(h1, s1, s2, s3, att_row, W1, b1, p0, p1):
    def body(h_r, s1l, s1h, s2l, s2h, s3l, s3h, a_r, w_r, b_r, p0_r, p1_r,
             h2_r, slo_r, shi_r):
        act, deg = _combine(h_r[...], [(s1l, s1h), (s2l, s2h), (s3l, s3h)],
                            a_r, p0_r, p1_r)
        h2 = jnp.dot(act, w_r[...], preferred_element_type=jnp.float32)
        h2 = h2 + b_r[...]
        h2_r[...] = h2
        s = h2 * lax.rsqrt(deg)
        slo_r[...] = s[:, :_HALF]
        shi_r[...] = s[:, _HALF:]

    return pl.pallas_call(
        body,
        grid=(_TCGRID,),
        in_specs=[
            _row_spec(_HID),
            _row_spec(_HALF), _row_spec(_HALF),
            _row_spec(_HALF), _row_spec(_HALF),
            _row_spec(_HALF), _row_spec(_HALF),
            pl.BlockSpec(memory_space=pltpu.SMEM),
            _full_spec((_HID, _HID)),
            _full_spec((1, _HID)),
            _row_spec(_DW),
            _row_spec(_DW),
        ],
        out_specs=[
            _row_spec(_HID),
            _row_spec(_HALF),
            _row_spec(_HALF),
        ],
        out_shape=[
            jax.ShapeDtypeStruct((_N, _HID), jnp.float32),
            jax.ShapeDtypeStruct((_N, _HALF), jnp.float32),
            jax.ShapeDtypeStruct((_N, _HALF), jnp.float32),
        ],
    )(h1, s1[0], s1[1], s2[0], s2[1], s3[0], s3[1],
      att_row.reshape(1, _K + 1), W1, b1.reshape(1, _HID), p0, p1)


def _tc_last(h2, s1, s2, s3, att_row, W2, b2, p0, p1):
    def body(h_r, s1l, s1h, s2l, s2h, s3l, s3h, a_r, w_r, b_r, p0_r, p1_r,
             out_r):
        act, _ = _combine(h_r[...], [(s1l, s1h), (s2l, s2h), (s3l, s3h)],
                          a_r, p0_r, p1_r)
        z = jnp.dot(act, w_r[...], preferred_element_type=jnp.float32)
        z = z + b_r[...]
        z = z - jnp.max(z, axis=1, keepdims=True)
        out_r[...] = z - jnp.log(jnp.sum(jnp.exp(z), axis=1, keepdims=True))

    return pl.pallas_call(
        body,
        grid=(_TCGRID,),
        in_specs=[
            _row_spec(_HID),
            _row_spec(_HALF), _row_spec(_HALF),
            _row_spec(_HALF), _row_spec(_HALF),
            _row_spec(_HALF), _row_spec(_HALF),
            pl.BlockSpec(memory_space=pltpu.SMEM),
            _full_spec((_HID, _NCLS)),
            _full_spec((1, _NCLS)),
            _row_spec(_DW),
            _row_spec(_DW),
        ],
        out_specs=_row_spec(_NCLS),
        out_shape=jax.ShapeDtypeStruct((_N, _NCLS), jnp.float32),
    )(h2, s1[0], s1[1], s2[0], s2[1], s3[0], s3[1],
      att_row.reshape(1, _K + 1), W2, b2.reshape(1, _NCLS), p0, p1)


def _pad_groups(idx, fill):
    pad = jnp.full((_EPAD - _E,), fill, jnp.int32)
    return jnp.concatenate([idx, pad]).reshape(_NG, 1, _G)


def kernel(x, edge_index, batch, att, W0, b0, W1, b1, W2, b2):
    del batch
    e0, e1 = edge_index[0], edge_index[1]
    row0 = _pad_groups(e0, _N)
    col0 = _pad_groups(e1, 0)
    row1 = _pad_groups(e1, _N)
    col1 = _pad_groups(e0, 0)

    pp = _sc_deg(row0, row1)
    p0, p1 = pp[:_N], pp[_N:]

    h1, slo, shi, rdeg = _tc_first(x, W0, b0, p0, p1)

    s1l, s1h, s2l, s2h, s3l, s3h = _sc_prop3(
        row0, col0, row1, col1, slo, shi, rdeg)

    h2, slo2, shi2 = _tc_mid(h1, (s1l, s1h), (s2l, s2h), (s3l, s3h),
                             att[0], W1, b1, p0, p1)

    t1l, t1h, t2l, t2h, t3l, t3h = _sc_prop3(
        row0, col0, row1, col1, slo2, shi2, rdeg)

    return _tc_last(h2, (t1l, t1h), (t2l, t2h), (t3l, t3h),
                    att[1], W2, b2, p0, p1)
